# Initial kernel scaffold; baseline (speedup 1.0000x reference)
import jax
import jax.numpy as jnp
from jax.experimental import pallas as pl


def kernel(x, edge_index, W1, b1, W2, b2, W3, b3):
    raise NotImplementedError("write your pallas kernel here")


# speedup vs baseline: 17.7982x; 17.7982x over previous
import functools

import jax
import jax.numpy as jnp
from jax import lax
from jax.experimental import pallas as pl
from jax.experimental.pallas import tpu as pltpu
from jax.experimental.pallas import tpu_sc as plsc

N = 10000
E = 320000
NC, NS = 2, 16
NW = NC * NS
GROUPS = 10
EPW = GROUPS * 8 * 128
EP = NW * EPW
IDX_ROWS = EP // 128
RPT = 624
TAIL_OFF = NS * RPT
TAIL = N - TAIL_OFF
BLK = 1000
GRID = N // BLK


def _sc_degree(dstm, ones, zeros8):
    mesh = plsc.VectorSubcoreMesh(core_axis_name="c", subcore_axis_name="s")

    @functools.partial(
        pl.kernel,
        out_type=jax.ShapeDtypeStruct((2 * N, 8), jnp.float32),
        mesh=mesh,
        scratch_types=[
            pltpu.VMEM((8, 128), jnp.int32),
            pltpu.VMEM((128, 8), jnp.float32),
            pltpu.VMEM_SHARED((N + 16, 8), jnp.float32),
        ],
    )
    def deg_kernel(dst_hbm, ones_hbm, z_hbm, out_hbm, dst_v, ones_v, acc):
        c = lax.axis_index("c")
        s = lax.axis_index("s")
        wid = s * NC + c
        off = pl.multiple_of(s * RPT, 8)
        pltpu.sync_copy(ones_hbm, ones_v)
        pltpu.sync_copy(z_hbm.at[pl.ds(off, RPT)], acc.at[pl.ds(off, RPT)])

        @pl.when(s == 0)
        def _():
            pltpu.sync_copy(z_hbm.at[pl.ds(TAIL_OFF, TAIL)],
                            acc.at[pl.ds(TAIL_OFF, TAIL)])

        plsc.subcore_barrier()

        row0 = wid * (GROUPS * 8)

        def group(gi, _):
            r = row0 + gi * 8
            pltpu.sync_copy(dst_hbm.at[pl.ds(r, 8)], dst_v)
            for j in range(8):
                pltpu.sync_copy(ones_v, acc.at[dst_v.at[j]], add=True)
            return ()

        lax.fori_loop(0, GROUPS, group, ())
        plsc.subcore_barrier()
        off2 = pl.multiple_of(c * N + s * RPT, 8)
        pltpu.sync_copy(acc.at[pl.ds(off, RPT)], out_hbm.at[pl.ds(off2, RPT)])

        @pl.when(s == 0)
        def _():
            off3 = pl.multiple_of(c * N + TAIL_OFF, 8)
            pltpu.sync_copy(acc.at[pl.ds(TAIL_OFF, TAIL)],
                            out_hbm.at[pl.ds(off3, TAIL)])

    return deg_kernel(dstm, ones, zeros8)


def _make_sc_agg(F):
    mesh = plsc.VectorSubcoreMesh(core_axis_name="c", subcore_axis_name="s")

    @functools.partial(
        pl.kernel,
        out_type=jax.ShapeDtypeStruct((2 * N, F), jnp.float32),
        mesh=mesh,
        scratch_types=[
            pltpu.VMEM((8, 128), jnp.int32),
            pltpu.VMEM((8, 128), jnp.int32),
            pltpu.VMEM((8, 128, F), jnp.float32),
            pltpu.VMEM_SHARED((N + 16, F), jnp.float32),
            pltpu.SemaphoreType.DMA,
        ],
        compiler_params=pltpu.CompilerParams(use_tc_tiling_on_sc=False),
    )
    def agg_kernel(g_hbm, z_hbm, src_hbm, dst_hbm, out_hbm,
                   src_v, dst_v, rows_v, acc, sem):
        c = lax.axis_index("c")
        s = lax.axis_index("s")
        wid = s * NC + c
        off = pl.multiple_of(s * RPT, 8)

        @pl.when(c == 0)
        def _():
            pltpu.sync_copy(g_hbm.at[pl.ds(off, RPT)], acc.at[pl.ds(off, RPT)])

            @pl.when(s == 0)
            def _():
                pltpu.sync_copy(g_hbm.at[pl.ds(TAIL_OFF, TAIL)],
                                acc.at[pl.ds(TAIL_OFF, TAIL)])

        @pl.when(c != 0)
        def _():
            pltpu.sync_copy(z_hbm.at[pl.ds(off, RPT)], acc.at[pl.ds(off, RPT)])

            @pl.when(s == 0)
            def _():
                pltpu.sync_copy(z_hbm.at[pl.ds(TAIL_OFF, TAIL)],
                                acc.at[pl.ds(TAIL_OFF, TAIL)])

        plsc.subcore_barrier()

        row0 = wid * (GROUPS * 8)

        def group(gi, _):
            r = row0 + gi * 8
            pltpu.sync_copy(src_hbm.at[pl.ds(r, 8)], src_v)
            pltpu.sync_copy(dst_hbm.at[pl.ds(r, 8)], dst_v)
            cps = [pltpu.async_copy(g_hbm.at[src_v.at[j]], rows_v.at[j], sem)
                   for j in range(8)]
            for cp in cps:
                cp.wait()
            for j in range(8):
                pltpu.sync_copy(rows_v.at[j], acc.at[dst_v.at[j]], add=True)
            return ()

        lax.fori_loop(0, GROUPS, group, ())
        plsc.subcore_barrier()
        off2 = pl.multiple_of(c * N + s * RPT, 8)
        pltpu.sync_copy(acc.at[pl.ds(off, RPT)], out_hbm.at[pl.ds(off2, RPT)])

        @pl.when(s == 0)
        def _():
            off3 = pl.multiple_of(c * N + TAIL_OFF, 8)
            pltpu.sync_copy(acc.at[pl.ds(TAIL_OFF, TAIL)],
                            out_hbm.at[pl.ds(off3, TAIL)])

    return agg_kernel


_sc_agg64 = _make_sc_agg(64)
_sc_agg32 = _make_sc_agg(32)


def _tc1_body(x_ref, w_ref, d0_ref, d1_ref, g_ref, dis_ref):
    deg = d0_ref[:, 0:1] + d1_ref[:, 0:1] + 1.0
    dis = lax.rsqrt(deg)
    dis_ref[...] = jnp.broadcast_to(dis, (BLK, 8))
    g_ref[...] = jnp.dot(x_ref[...], w_ref[...],
                         preferred_element_type=jnp.float32) * dis


def _tc1(x, W1, degp):
    return pl.pallas_call(
        _tc1_body,
        grid=(GRID,),
        in_specs=[
            pl.BlockSpec((BLK, 128), lambda i: (i, 0)),
            pl.BlockSpec((128, 64), lambda i: (0, 0)),
            pl.BlockSpec((BLK, 8), lambda i: (i, 0)),
            pl.BlockSpec((BLK, 8), lambda i: (i + GRID, 0)),
        ],
        out_specs=[
            pl.BlockSpec((BLK, 64), lambda i: (i, 0)),
            pl.BlockSpec((BLK, 8), lambda i: (i, 0)),
        ],
        out_shape=[
            jax.ShapeDtypeStruct((N, 64), jnp.float32),
            jax.ShapeDtypeStruct((N, 8), jnp.float32),
        ],
    )(x, W1, degp, degp)


def _tc2_body(a0_ref, a1_ref, dis_ref, b_ref, w_ref, g_ref):
    d = dis_ref[:, 0:1]
    h = jnp.maximum((a0_ref[...] + a1_ref[...]) * d + b_ref[...], 0.0)
    g_ref[...] = jnp.dot(h, w_ref[...], preferred_element_type=jnp.float32) * d


def _tc2(agg1, dis, b1r, W2):
    return pl.pallas_call(
        _tc2_body,
        grid=(GRID,),
        in_specs=[
            pl.BlockSpec((BLK, 64), lambda i: (i, 0)),
            pl.BlockSpec((BLK, 64), lambda i: (i + GRID, 0)),
            pl.BlockSpec((BLK, 8), lambda i: (i, 0)),
            pl.BlockSpec((1, 64), lambda i: (0, 0)),
            pl.BlockSpec((64, 32), lambda i: (0, 0)),
        ],
        out_specs=pl.BlockSpec((BLK, 32), lambda i: (i, 0)),
        out_shape=jax.ShapeDtypeStruct((N, 32), jnp.float32),
    )(agg1, agg1, dis, b1r, W2)


def _tc3_body(a0_ref, a1_ref, dis_ref, b2_ref, w_ref, b3_ref, o_ref):
    d = dis_ref[:, 0:1]
    h = jnp.maximum((a0_ref[...] + a1_ref[...]) * d + b2_ref[...], 0.0)
    o_ref[...] = jnp.dot(h, w_ref[...],
                         preferred_element_type=jnp.float32) + b3_ref[...]


def _tc3(agg2, dis, b2r, W3, b3r):
    return pl.pallas_call(
        _tc3_body,
        grid=(GRID,),
        in_specs=[
            pl.BlockSpec((BLK, 32), lambda i: (i, 0)),
            pl.BlockSpec((BLK, 32), lambda i: (i + GRID, 0)),
            pl.BlockSpec((BLK, 8), lambda i: (i, 0)),
            pl.BlockSpec((1, 32), lambda i: (0, 0)),
            pl.BlockSpec((32, 1), lambda i: (0, 0)),
            pl.BlockSpec((1, 1), lambda i: (0, 0)),
        ],
        out_specs=pl.BlockSpec((BLK, 1), lambda i: (i, 0)),
        out_shape=jax.ShapeDtypeStruct((N, 1), jnp.float32),
    )(agg2, agg2, dis, b2r, W3, b3r)


def kernel(x, edge_index, W1, b1, W2, b2, W3, b3):
    ei = edge_index.astype(jnp.int32)
    src = jnp.concatenate(
        [ei[0], jnp.zeros((EP - E,), jnp.int32)]).reshape(IDX_ROWS, 128)
    dst = jnp.concatenate(
        [ei[1], jnp.full((EP - E,), N, jnp.int32)]).reshape(IDX_ROWS, 128)
    ones = jnp.ones((128, 8), jnp.float32)
    z8 = jnp.zeros((N, 8), jnp.float32)
    z64 = jnp.zeros((N, 64), jnp.float32)
    z32 = jnp.zeros((N, 32), jnp.float32)

    degp = _sc_degree(dst, ones, z8)
    g1, dis = _tc1(x, W1, degp)
    agg1 = _sc_agg64(g1, z64, src, dst)
    g2 = _tc2(agg1, dis, b1.reshape(1, 64), W2)
    agg2 = _sc_agg32(g2, z32, src, dst)
    out = _tc3(agg2, dis, b2.reshape(1, 32), W3, b3.reshape(1, 1))
    return out


# speedup vs baseline: 26.5012x; 1.4890x over previous
import functools

import jax
import jax.numpy as jnp
from jax import lax
from jax.experimental import pallas as pl
from jax.experimental.pallas import tpu as pltpu
from jax.experimental.pallas import tpu_sc as plsc

N = 10000
E = 320000
NC, NS = 2, 16
IDX_ROWS = E // 128
GRP = 4
R0, R1 = 112, 44
XTRA_ROW = NS * (R0 + R1)
RPT = 624
TAIL_OFF = NS * RPT
TAIL = N - TAIL_OFF
BLK = 2000
GRID = N // BLK

_SC_PARAMS = pltpu.CompilerParams(use_tc_tiling_on_sc=False)


def _sc_degree(dstm, ones, zeros8):
    mesh = plsc.VectorSubcoreMesh(core_axis_name="c", subcore_axis_name="s")

    @functools.partial(
        pl.kernel,
        out_type=jax.ShapeDtypeStruct((2 * N, 8), jnp.float32),
        mesh=mesh,
        scratch_types=[
            pltpu.VMEM((GRP, 128), jnp.int32),
            pltpu.VMEM((128, 8), jnp.float32),
            pltpu.VMEM_SHARED((N + 16, 8), jnp.float32),
        ],
        compiler_params=_SC_PARAMS,
    )
    def deg_kernel(dst_hbm, ones_hbm, z_hbm, out_hbm, dst_v, ones_v, acc):
        c = lax.axis_index("c")
        s = lax.axis_index("s")
        wid = s * NC + c
        off = pl.multiple_of(s * RPT, 8)
        pltpu.sync_copy(ones_hbm, ones_v)
        pltpu.sync_copy(z_hbm.at[pl.ds(off, RPT)], acc.at[pl.ds(off, RPT)])

        @pl.when(s == 0)
        def _():
            pltpu.sync_copy(z_hbm.at[pl.ds(TAIL_OFF, TAIL)],
                            acc.at[pl.ds(TAIL_OFF, TAIL)])

        plsc.subcore_barrier()

        row0 = wid * 78

        def group(gi, _):
            r = pl.multiple_of(row0 + gi * GRP, 2)
            pltpu.sync_copy(dst_hbm.at[pl.ds(r, GRP)], dst_v)
            for j in range(GRP):
                pltpu.sync_copy(ones_v, acc.at[dst_v.at[j]], add=True)
            return ()

        lax.fori_loop(0, 19, group, ())
        r2 = pl.multiple_of(row0 + 76, 2)
        pltpu.sync_copy(dst_hbm.at[pl.ds(r2, 2)], dst_v.at[pl.ds(0, 2)])
        for j in range(2):
            pltpu.sync_copy(ones_v, acc.at[dst_v.at[j]], add=True)

        @pl.when((c == 0) & (s == 0))
        def _():
            pltpu.sync_copy(dst_hbm.at[pl.ds(XTRA_ROW, 4)],
                            dst_v.at[pl.ds(0, 4)])
            for j in range(4):
                pltpu.sync_copy(ones_v, acc.at[dst_v.at[j]], add=True)

        plsc.subcore_barrier()
        off2 = pl.multiple_of(c * N + s * RPT, 8)
        pltpu.sync_copy(acc.at[pl.ds(off, RPT)], out_hbm.at[pl.ds(off2, RPT)])

        @pl.when(s == 0)
        def _():
            off3 = pl.multiple_of(c * N + TAIL_OFF, 8)
            pltpu.sync_copy(acc.at[pl.ds(TAIL_OFF, TAIL)],
                            out_hbm.at[pl.ds(off3, TAIL)])

    return deg_kernel(dstm, ones, zeros8)


def _make_sc_agg(F):
    mesh = plsc.VectorSubcoreMesh(core_axis_name="c", subcore_axis_name="s")

    @functools.partial(
        pl.kernel,
        out_type=jax.ShapeDtypeStruct((2 * N, F), jnp.float32),
        mesh=mesh,
        scratch_types=[
            pltpu.VMEM((GRP, 128), jnp.int32),
            pltpu.VMEM((GRP, 128), jnp.int32),
            pltpu.VMEM((GRP, 128, F), jnp.float32),
            pltpu.VMEM_SHARED((N + 16, F), jnp.float32),
            pltpu.SemaphoreType.DMA,
        ],
        compiler_params=_SC_PARAMS,
    )
    def agg_kernel(g_hbm, z_hbm, src_hbm, dst_hbm, out_hbm,
                   src_v, dst_v, rows_v, acc, sem):
        c = lax.axis_index("c")
        s = lax.axis_index("s")
        off = pl.multiple_of(s * RPT, 8)

        @pl.when(c == 0)
        def _():
            pltpu.sync_copy(g_hbm.at[pl.ds(off, RPT)], acc.at[pl.ds(off, RPT)])

            @pl.when(s == 0)
            def _():
                pltpu.sync_copy(g_hbm.at[pl.ds(TAIL_OFF, TAIL)],
                                acc.at[pl.ds(TAIL_OFF, TAIL)])

        @pl.when(c != 0)
        def _():
            pltpu.sync_copy(z_hbm.at[pl.ds(off, RPT)], acc.at[pl.ds(off, RPT)])

            @pl.when(s == 0)
            def _():
                pltpu.sync_copy(z_hbm.at[pl.ds(TAIL_OFF, TAIL)],
                                acc.at[pl.ds(TAIL_OFF, TAIL)])

        plsc.subcore_barrier()

        row0 = jnp.where(c == 0, s * R0, NS * R0 + s * R1)
        ngroups = jnp.where(c == 0, R0 // GRP, R1 // GRP)

        def group(gi, _):
            r = pl.multiple_of(row0 + gi * GRP, 4)
            pltpu.sync_copy(src_hbm.at[pl.ds(r, GRP)], src_v)
            pltpu.sync_copy(dst_hbm.at[pl.ds(r, GRP)], dst_v)
            cps = [pltpu.async_copy(g_hbm.at[src_v.at[j]], rows_v.at[j], sem)
                   for j in range(GRP)]
            for cp in cps:
                cp.wait()
            for j in range(GRP):
                pltpu.sync_copy(rows_v.at[j], acc.at[dst_v.at[j]], add=True)
            return ()

        lax.fori_loop(0, ngroups, group, ())

        @pl.when((c == 0) & (s == 0))
        def _():
            pltpu.sync_copy(src_hbm.at[pl.ds(XTRA_ROW, 4)], src_v)
            pltpu.sync_copy(dst_hbm.at[pl.ds(XTRA_ROW, 4)], dst_v)
            cps = [pltpu.async_copy(g_hbm.at[src_v.at[j]], rows_v.at[j], sem)
                   for j in range(4)]
            for cp in cps:
                cp.wait()
            for j in range(4):
                pltpu.sync_copy(rows_v.at[j], acc.at[dst_v.at[j]], add=True)

        plsc.subcore_barrier()
        off2 = pl.multiple_of(c * N + s * RPT, 8)
        pltpu.sync_copy(acc.at[pl.ds(off, RPT)], out_hbm.at[pl.ds(off2, RPT)])

        @pl.when(s == 0)
        def _():
            off3 = pl.multiple_of(c * N + TAIL_OFF, 8)
            pltpu.sync_copy(acc.at[pl.ds(TAIL_OFF, TAIL)],
                            out_hbm.at[pl.ds(off3, TAIL)])

    return agg_kernel


_sc_agg64 = _make_sc_agg(64)
_sc_agg32 = _make_sc_agg(32)


def _tc1_body(x_ref, w_ref, d0_ref, d1_ref, g_ref, dis_ref):
    deg = d0_ref[:, 0:1] + d1_ref[:, 0:1] + 1.0
    dis = lax.rsqrt(deg)
    dis_ref[...] = jnp.broadcast_to(dis, (BLK, 8))
    g_ref[...] = jnp.dot(x_ref[...], w_ref[...],
                         preferred_element_type=jnp.float32) * dis


def _tc1(x, W1, degp):
    return pl.pallas_call(
        _tc1_body,
        grid=(GRID,),
        in_specs=[
            pl.BlockSpec((BLK, 128), lambda i: (i, 0)),
            pl.BlockSpec((128, 64), lambda i: (0, 0)),
            pl.BlockSpec((BLK, 8), lambda i: (i, 0)),
            pl.BlockSpec((BLK, 8), lambda i: (i + GRID, 0)),
        ],
        out_specs=[
            pl.BlockSpec((BLK, 64), lambda i: (i, 0)),
            pl.BlockSpec((BLK, 8), lambda i: (i, 0)),
        ],
        out_shape=[
            jax.ShapeDtypeStruct((N, 64), jnp.float32),
            jax.ShapeDtypeStruct((N, 8), jnp.float32),
        ],
    )(x, W1, degp, degp)


def _tc2_body(a0_ref, a1_ref, dis_ref, b_ref, w_ref, g_ref):
    d = dis_ref[:, 0:1]
    h = jnp.maximum((a0_ref[...] + a1_ref[...]) * d + b_ref[...], 0.0)
    g_ref[...] = jnp.dot(h, w_ref[...], preferred_element_type=jnp.float32) * d


def _tc2(agg1, dis, b1r, W2):
    return pl.pallas_call(
        _tc2_body,
        grid=(GRID,),
        in_specs=[
            pl.BlockSpec((BLK, 64), lambda i: (i, 0)),
            pl.BlockSpec((BLK, 64), lambda i: (i + GRID, 0)),
            pl.BlockSpec((BLK, 8), lambda i: (i, 0)),
            pl.BlockSpec((1, 64), lambda i: (0, 0)),
            pl.BlockSpec((64, 32), lambda i: (0, 0)),
        ],
        out_specs=pl.BlockSpec((BLK, 32), lambda i: (i, 0)),
        out_shape=jax.ShapeDtypeStruct((N, 32), jnp.float32),
    )(agg1, agg1, dis, b1r, W2)


def _tc3_body(a0_ref, a1_ref, dis_ref, b2_ref, w_ref, b3_ref, o_ref):
    d = dis_ref[:, 0:1]
    h = jnp.maximum((a0_ref[...] + a1_ref[...]) * d + b2_ref[...], 0.0)
    o_ref[...] = jnp.dot(h, w_ref[...],
                         preferred_element_type=jnp.float32) + b3_ref[...]


def _tc3(agg2, dis, b2r, W3, b3r):
    return pl.pallas_call(
        _tc3_body,
        grid=(GRID,),
        in_specs=[
            pl.BlockSpec((BLK, 32), lambda i: (i, 0)),
            pl.BlockSpec((BLK, 32), lambda i: (i + GRID, 0)),
            pl.BlockSpec((BLK, 8), lambda i: (i, 0)),
            pl.BlockSpec((1, 32), lambda i: (0, 0)),
            pl.BlockSpec((32, 1), lambda i: (0, 0)),
            pl.BlockSpec((1, 1), lambda i: (0, 0)),
        ],
        out_specs=pl.BlockSpec((BLK, 1), lambda i: (i, 0)),
        out_shape=jax.ShapeDtypeStruct((N, 1), jnp.float32),
    )(agg2, agg2, dis, b2r, W3, b3r)


def kernel(x, edge_index, W1, b1, W2, b2, W3, b3):
    ei = edge_index.astype(jnp.int32)
    src = ei[0].reshape(IDX_ROWS, 128)
    dst = ei[1].reshape(IDX_ROWS, 128)
    ones = jnp.ones((128, 8), jnp.float32)
    z8 = jnp.zeros((N, 8), jnp.float32)
    z64 = jnp.zeros((N, 64), jnp.float32)
    z32 = jnp.zeros((N, 32), jnp.float32)

    degp = _sc_degree(dst, ones, z8)
    g1, dis = _tc1(x, W1, degp)
    agg1 = _sc_agg64(g1, z64, src, dst)
    g2 = _tc2(agg1, dis, b1.reshape(1, 64), W2)
    agg2 = _sc_agg32(g2, z32, src, dst)
    out = _tc3(agg2, dis, b2.reshape(1, 32), W3, b3.reshape(1, 1))
    return out


# speedup vs baseline: 37.4975x; 1.4149x over previous
import functools

import jax
import jax.numpy as jnp
from jax import lax
from jax.experimental import pallas as pl
from jax.experimental.pallas import tpu as pltpu
from jax.experimental.pallas import tpu_sc as plsc

N = 10000
E = 320000
NC, NS = 2, 16
IDX_ROWS = E // 128
GRP = 4
R0, R1 = 88, 68
XTRA_ROW = NS * (R0 + R1)
RPT = 624
TAIL_OFF = NS * RPT
TAIL = N - TAIL_OFF
BLK = 2000
GRID = N // BLK

_SC_PARAMS = pltpu.CompilerParams(use_tc_tiling_on_sc=False)


def _sc_degree(dstm, ones, zeros8):
    mesh = plsc.VectorSubcoreMesh(core_axis_name="c", subcore_axis_name="s")

    @functools.partial(
        pl.kernel,
        out_type=jax.ShapeDtypeStruct((2 * N, 8), jnp.float32),
        mesh=mesh,
        scratch_types=[
            pltpu.VMEM((GRP, 128), jnp.int32),
            pltpu.VMEM((128, 8), jnp.float32),
            pltpu.VMEM_SHARED((N + 16, 8), jnp.float32),
        ],
        compiler_params=_SC_PARAMS,
    )
    def deg_kernel(dst_hbm, ones_hbm, z_hbm, out_hbm, dst_v, ones_v, acc):
        c = lax.axis_index("c")
        s = lax.axis_index("s")
        wid = s * NC + c
        off = pl.multiple_of(s * RPT, 8)
        pltpu.sync_copy(ones_hbm, ones_v)
        pltpu.sync_copy(z_hbm.at[pl.ds(off, RPT)], acc.at[pl.ds(off, RPT)])

        @pl.when(s == 0)
        def _():
            pltpu.sync_copy(z_hbm.at[pl.ds(TAIL_OFF, TAIL)],
                            acc.at[pl.ds(TAIL_OFF, TAIL)])

        plsc.subcore_barrier()

        row0 = wid * 78

        def group(gi, _):
            r = pl.multiple_of(row0 + gi * GRP, 2)
            pltpu.sync_copy(dst_hbm.at[pl.ds(r, GRP)], dst_v)
            for j in range(GRP):
                pltpu.sync_copy(ones_v, acc.at[dst_v.at[j]], add=True)
            return ()

        lax.fori_loop(0, 19, group, ())
        r2 = pl.multiple_of(row0 + 76, 2)
        pltpu.sync_copy(dst_hbm.at[pl.ds(r2, 2)], dst_v.at[pl.ds(0, 2)])
        for j in range(2):
            pltpu.sync_copy(ones_v, acc.at[dst_v.at[j]], add=True)

        @pl.when((c == 0) & (s == 0))
        def _():
            pltpu.sync_copy(dst_hbm.at[pl.ds(XTRA_ROW, 4)],
                            dst_v.at[pl.ds(0, 4)])
            for j in range(4):
                pltpu.sync_copy(ones_v, acc.at[dst_v.at[j]], add=True)

        plsc.subcore_barrier()
        off2 = pl.multiple_of(c * N + s * RPT, 8)
        pltpu.sync_copy(acc.at[pl.ds(off, RPT)], out_hbm.at[pl.ds(off2, RPT)])

        @pl.when(s == 0)
        def _():
            off3 = pl.multiple_of(c * N + TAIL_OFF, 8)
            pltpu.sync_copy(acc.at[pl.ds(TAIL_OFF, TAIL)],
                            out_hbm.at[pl.ds(off3, TAIL)])

    return deg_kernel(dstm, ones, zeros8)


def _make_sc_agg(F):
    mesh = plsc.VectorSubcoreMesh(core_axis_name="c", subcore_axis_name="s")

    @functools.partial(
        pl.kernel,
        out_type=jax.ShapeDtypeStruct((2 * N, F), jnp.float32),
        mesh=mesh,
        scratch_types=[
            pltpu.VMEM((GRP, 128), jnp.int32),
            pltpu.VMEM((GRP, 128), jnp.int32),
            pltpu.VMEM((GRP, 128), jnp.int32),
            pltpu.VMEM((GRP, 128), jnp.int32),
            pltpu.VMEM((GRP, 128, F), jnp.float32),
            pltpu.VMEM((GRP, 128, F), jnp.float32),
            pltpu.VMEM_SHARED((N + 16, F), jnp.float32),
            pltpu.SemaphoreType.DMA,
        ],
        compiler_params=_SC_PARAMS,
    )
    def agg_kernel(g_hbm, z_hbm, src_hbm, dst_hbm, out_hbm,
                   src_a, dst_a, src_b, dst_b, rows_a, rows_b, acc, sem):
        c = lax.axis_index("c")
        s = lax.axis_index("s")
        off = pl.multiple_of(s * RPT, 8)

        @pl.when(c == 0)
        def _():
            pltpu.sync_copy(g_hbm.at[pl.ds(off, RPT)], acc.at[pl.ds(off, RPT)])

            @pl.when(s == 0)
            def _():
                pltpu.sync_copy(g_hbm.at[pl.ds(TAIL_OFF, TAIL)],
                                acc.at[pl.ds(TAIL_OFF, TAIL)])

        @pl.when(c != 0)
        def _():
            pltpu.sync_copy(z_hbm.at[pl.ds(off, RPT)], acc.at[pl.ds(off, RPT)])

            @pl.when(s == 0)
            def _():
                pltpu.sync_copy(z_hbm.at[pl.ds(TAIL_OFF, TAIL)],
                                acc.at[pl.ds(TAIL_OFF, TAIL)])

        plsc.subcore_barrier()

        row0 = jnp.where(c == 0, s * R0, NS * R0 + s * R1)
        ngroups = jnp.where(c == 0, R0 // GRP, R1 // GRP)

        def fire(gi, src_v, dst_v, rows_v):
            r = pl.multiple_of(row0 + gi * GRP, 4)
            pltpu.sync_copy(src_hbm.at[pl.ds(r, GRP)], src_v)
            pltpu.sync_copy(dst_hbm.at[pl.ds(r, GRP)], dst_v)
            for j in range(GRP):
                pltpu.async_copy(g_hbm.at[src_v.at[j]], rows_v.at[j], sem)

        def drain_scatter(dst_v, rows_v):
            for j in range(GRP):
                pltpu.make_async_copy(g_hbm.at[pl.ds(0, 128)],
                                      rows_v.at[j], sem).wait()
            for j in range(GRP):
                pltpu.sync_copy(rows_v.at[j], acc.at[dst_v.at[j]], add=True)

        def group(gi, _):
            even = (gi % 2) == 0

            @pl.when(even)
            def _():
                fire(gi, src_a, dst_a, rows_a)

            @pl.when(jnp.logical_not(even))
            def _():
                fire(gi, src_b, dst_b, rows_b)

            @pl.when((gi > 0) & even)
            def _():
                drain_scatter(dst_b, rows_b)

            @pl.when((gi > 0) & jnp.logical_not(even))
            def _():
                drain_scatter(dst_a, rows_a)

            return ()

        lax.fori_loop(0, ngroups, group, ())
        last_even = ((ngroups - 1) % 2) == 0

        @pl.when(last_even)
        def _():
            drain_scatter(dst_a, rows_a)

        @pl.when(jnp.logical_not(last_even))
        def _():
            drain_scatter(dst_b, rows_b)

        @pl.when((c == 0) & (s == 0))
        def _():
            fire_r = pl.multiple_of(XTRA_ROW, 4)
            pltpu.sync_copy(src_hbm.at[pl.ds(fire_r, 4)], src_a)
            pltpu.sync_copy(dst_hbm.at[pl.ds(fire_r, 4)], dst_a)
            for j in range(4):
                pltpu.async_copy(g_hbm.at[src_a.at[j]], rows_a.at[j], sem)
            drain_scatter(dst_a, rows_a)

        plsc.subcore_barrier()
        off2 = pl.multiple_of(c * N + s * RPT, 8)
        pltpu.sync_copy(acc.at[pl.ds(off, RPT)], out_hbm.at[pl.ds(off2, RPT)])

        @pl.when(s == 0)
        def _():
            off3 = pl.multiple_of(c * N + TAIL_OFF, 8)
            pltpu.sync_copy(acc.at[pl.ds(TAIL_OFF, TAIL)],
                            out_hbm.at[pl.ds(off3, TAIL)])

    return agg_kernel


_sc_agg64 = _make_sc_agg(64)
_sc_agg32 = _make_sc_agg(32)


def _tc1_body(x_ref, w_ref, d0_ref, d1_ref, g_ref, dis_ref):
    deg = d0_ref[:, 0:1] + d1_ref[:, 0:1] + 1.0
    dis = lax.rsqrt(deg)
    dis_ref[...] = jnp.broadcast_to(dis, (BLK, 8))
    g_ref[...] = jnp.dot(x_ref[...], w_ref[...],
                         preferred_element_type=jnp.float32) * dis


def _tc1(x, W1, degp):
    return pl.pallas_call(
        _tc1_body,
        grid=(GRID,),
        in_specs=[
            pl.BlockSpec((BLK, 128), lambda i: (i, 0)),
            pl.BlockSpec((128, 64), lambda i: (0, 0)),
            pl.BlockSpec((BLK, 8), lambda i: (i, 0)),
            pl.BlockSpec((BLK, 8), lambda i: (i + GRID, 0)),
        ],
        out_specs=[
            pl.BlockSpec((BLK, 64), lambda i: (i, 0)),
            pl.BlockSpec((BLK, 8), lambda i: (i, 0)),
        ],
        out_shape=[
            jax.ShapeDtypeStruct((N, 64), jnp.float32),
            jax.ShapeDtypeStruct((N, 8), jnp.float32),
        ],
    )(x, W1, degp, degp)


def _tc2_body(a0_ref, a1_ref, dis_ref, b_ref, w_ref, g_ref):
    d = dis_ref[:, 0:1]
    h = jnp.maximum((a0_ref[...] + a1_ref[...]) * d + b_ref[...], 0.0)
    g_ref[...] = jnp.dot(h, w_ref[...], preferred_element_type=jnp.float32) * d


def _tc2(agg1, dis, b1r, W2):
    return pl.pallas_call(
        _tc2_body,
        grid=(GRID,),
        in_specs=[
            pl.BlockSpec((BLK, 64), lambda i: (i, 0)),
            pl.BlockSpec((BLK, 64), lambda i: (i + GRID, 0)),
            pl.BlockSpec((BLK, 8), lambda i: (i, 0)),
            pl.BlockSpec((1, 64), lambda i: (0, 0)),
            pl.BlockSpec((64, 32), lambda i: (0, 0)),
        ],
        out_specs=pl.BlockSpec((BLK, 32), lambda i: (i, 0)),
        out_shape=jax.ShapeDtypeStruct((N, 32), jnp.float32),
    )(agg1, agg1, dis, b1r, W2)


def _tc3_body(a0_ref, a1_ref, dis_ref, b2_ref, w_ref, b3_ref, o_ref):
    d = dis_ref[:, 0:1]
    h = jnp.maximum((a0_ref[...] + a1_ref[...]) * d + b2_ref[...], 0.0)
    o_ref[...] = jnp.dot(h, w_ref[...],
                         preferred_element_type=jnp.float32) + b3_ref[...]


def _tc3(agg2, dis, b2r, W3, b3r):
    return pl.pallas_call(
        _tc3_body,
        grid=(GRID,),
        in_specs=[
            pl.BlockSpec((BLK, 32), lambda i: (i, 0)),
            pl.BlockSpec((BLK, 32), lambda i: (i + GRID, 0)),
            pl.BlockSpec((BLK, 8), lambda i: (i, 0)),
            pl.BlockSpec((1, 32), lambda i: (0, 0)),
            pl.BlockSpec((32, 1), lambda i: (0, 0)),
            pl.BlockSpec((1, 1), lambda i: (0, 0)),
        ],
        out_specs=pl.BlockSpec((BLK, 1), lambda i: (i, 0)),
        out_shape=jax.ShapeDtypeStruct((N, 1), jnp.float32),
    )(agg2, agg2, dis, b2r, W3, b3r)


def kernel(x, edge_index, W1, b1, W2, b2, W3, b3):
    ei = edge_index.astype(jnp.int32)
    src = ei[0].reshape(IDX_ROWS, 128)
    dst = ei[1].reshape(IDX_ROWS, 128)
    ones = jnp.ones((128, 8), jnp.float32)
    z8 = jnp.zeros((N, 8), jnp.float32)
    z64 = jnp.zeros((N, 64), jnp.float32)
    z32 = jnp.zeros((N, 32), jnp.float32)

    degp = _sc_degree(dst, ones, z8)
    g1, dis = _tc1(x, W1, degp)
    agg1 = _sc_agg64(g1, z64, src, dst)
    g2 = _tc2(agg1, dis, b1.reshape(1, 64), W2)
    agg2 = _sc_agg32(g2, z32, src, dst)
    out = _tc3(agg2, dis, b2.reshape(1, 32), W3, b3.reshape(1, 1))
    return out


# speedup vs baseline: 40.1245x; 1.0701x over previous
import functools

import jax
import jax.numpy as jnp
from jax import lax
from jax.experimental import pallas as pl
from jax.experimental.pallas import tpu as pltpu
from jax.experimental.pallas import tpu_sc as plsc

N = 10000
E = 320000
NC, NS = 2, 16
IDX_ROWS = E // 128
GRP = 4
R0, R1 = 80, 76
XTRA_ROW = NS * (R0 + R1)
RPT = 624
TAIL_OFF = NS * RPT
TAIL = N - TAIL_OFF
DEG_R = 78

_SC_PARAMS = pltpu.CompilerParams(use_tc_tiling_on_sc=False)


def _sc_degree(dstm, ones, zeros8):
    mesh = plsc.VectorSubcoreMesh(core_axis_name="c", subcore_axis_name="s")

    @functools.partial(
        pl.kernel,
        out_type=jax.ShapeDtypeStruct((2 * N, 8), jnp.float32),
        mesh=mesh,
        scratch_types=[
            pltpu.VMEM((GRP, 128), jnp.int32),
            pltpu.VMEM((GRP, 128), jnp.int32),
            pltpu.VMEM((128, 8), jnp.float32),
            pltpu.VMEM_SHARED((N + 16, 8), jnp.float32),
            pltpu.SemaphoreType.DMA,
        ],
        compiler_params=_SC_PARAMS,
    )
    def deg_kernel(dst_hbm, ones_hbm, z_hbm, out_hbm,
                   dst_a, dst_b, ones_v, acc, sem):
        c = lax.axis_index("c")
        s = lax.axis_index("s")
        wid = s * NC + c
        off = pl.multiple_of(s * RPT, 8)
        pltpu.sync_copy(ones_hbm, ones_v)
        pltpu.sync_copy(z_hbm.at[pl.ds(off, RPT)], acc.at[pl.ds(off, RPT)])

        @pl.when(s == 0)
        def _():
            pltpu.sync_copy(z_hbm.at[pl.ds(TAIL_OFF, TAIL)],
                            acc.at[pl.ds(TAIL_OFF, TAIL)])

        plsc.subcore_barrier()

        row0 = wid * DEG_R

        def fire(gi, dst_v):
            r = pl.multiple_of(row0 + gi * GRP, 2)
            pltpu.async_copy(dst_hbm.at[pl.ds(r, GRP)], dst_v, sem)

        def drain_scatter(dst_v, nrows):
            pltpu.make_async_copy(dst_hbm.at[pl.ds(0, GRP)], dst_v, sem).wait()
            for j in range(nrows):
                pltpu.sync_copy(ones_v, acc.at[dst_v.at[j]], add=True)

        def group(gi, _):
            even = (gi % 2) == 0

            @pl.when(even)
            def _():
                fire(gi, dst_a)

            @pl.when(jnp.logical_not(even))
            def _():
                fire(gi, dst_b)

            @pl.when((gi > 0) & even)
            def _():
                drain_scatter(dst_b, GRP)

            @pl.when((gi > 0) & jnp.logical_not(even))
            def _():
                drain_scatter(dst_a, GRP)

            return ()

        lax.fori_loop(0, 19, group, ())
        drain_scatter(dst_a, GRP)
        r2 = pl.multiple_of(row0 + 76, 2)
        pltpu.sync_copy(dst_hbm.at[pl.ds(r2, 2)], dst_a.at[pl.ds(0, 2)])
        for j in range(2):
            pltpu.sync_copy(ones_v, acc.at[dst_a.at[j]], add=True)

        @pl.when((c == 0) & (s == 0))
        def _():
            pltpu.sync_copy(dst_hbm.at[pl.ds(XTRA_ROW, 4)], dst_a)
            for j in range(4):
                pltpu.sync_copy(ones_v, acc.at[dst_a.at[j]], add=True)

        plsc.subcore_barrier()
        off2 = pl.multiple_of(c * N + s * RPT, 8)
        pltpu.sync_copy(acc.at[pl.ds(off, RPT)], out_hbm.at[pl.ds(off2, RPT)])

        @pl.when(s == 0)
        def _():
            off3 = pl.multiple_of(c * N + TAIL_OFF, 8)
            pltpu.sync_copy(acc.at[pl.ds(TAIL_OFF, TAIL)],
                            out_hbm.at[pl.ds(off3, TAIL)])

    return deg_kernel(dstm, ones, zeros8)


def _make_sc_agg(F):
    mesh = plsc.VectorSubcoreMesh(core_axis_name="c", subcore_axis_name="s")

    @functools.partial(
        pl.kernel,
        out_type=jax.ShapeDtypeStruct((2 * N, F), jnp.float32),
        mesh=mesh,
        scratch_types=[
            pltpu.VMEM((GRP, 128), jnp.int32),
            pltpu.VMEM((GRP, 128), jnp.int32),
            pltpu.VMEM((GRP, 128), jnp.int32),
            pltpu.VMEM((GRP, 128), jnp.int32),
            pltpu.VMEM((GRP, 128, F), jnp.float32),
            pltpu.VMEM((GRP, 128, F), jnp.float32),
            pltpu.VMEM_SHARED((N + 16, F), jnp.float32),
            pltpu.SemaphoreType.DMA,
        ],
        compiler_params=_SC_PARAMS,
    )
    def agg_kernel(g_hbm, z_hbm, src_hbm, dst_hbm, out_hbm,
                   src_a, dst_a, src_b, dst_b, rows_a, rows_b, acc, sem):
        c = lax.axis_index("c")
        s = lax.axis_index("s")
        off = pl.multiple_of(s * RPT, 8)

        @pl.when(c == 0)
        def _():
            pltpu.sync_copy(g_hbm.at[pl.ds(off, RPT)], acc.at[pl.ds(off, RPT)])

            @pl.when(s == 0)
            def _():
                pltpu.sync_copy(g_hbm.at[pl.ds(TAIL_OFF, TAIL)],
                                acc.at[pl.ds(TAIL_OFF, TAIL)])

        @pl.when(c != 0)
        def _():
            pltpu.sync_copy(z_hbm.at[pl.ds(off, RPT)], acc.at[pl.ds(off, RPT)])

            @pl.when(s == 0)
            def _():
                pltpu.sync_copy(z_hbm.at[pl.ds(TAIL_OFF, TAIL)],
                                acc.at[pl.ds(TAIL_OFF, TAIL)])

        plsc.subcore_barrier()

        row0 = jnp.where(c == 0, s * R0, NS * R0 + s * R1)
        ngroups = jnp.where(c == 0, R0 // GRP, R1 // GRP)

        def fire(gi, src_v, dst_v, rows_v):
            r = pl.multiple_of(row0 + gi * GRP, 4)
            pltpu.sync_copy(src_hbm.at[pl.ds(r, GRP)], src_v)
            pltpu.sync_copy(dst_hbm.at[pl.ds(r, GRP)], dst_v)
            for j in range(GRP):
                pltpu.async_copy(g_hbm.at[src_v.at[j]], rows_v.at[j], sem)

        def drain_scatter(dst_v, rows_v):
            for j in range(GRP):
                pltpu.make_async_copy(g_hbm.at[pl.ds(0, 128)],
                                      rows_v.at[j], sem).wait()
            for j in range(GRP):
                pltpu.sync_copy(rows_v.at[j], acc.at[dst_v.at[j]], add=True)

        def group(gi, _):
            even = (gi % 2) == 0

            @pl.when(even)
            def _():
                fire(gi, src_a, dst_a, rows_a)

            @pl.when(jnp.logical_not(even))
            def _():
                fire(gi, src_b, dst_b, rows_b)

            @pl.when((gi > 0) & even)
            def _():
                drain_scatter(dst_b, rows_b)

            @pl.when((gi > 0) & jnp.logical_not(even))
            def _():
                drain_scatter(dst_a, rows_a)

            return ()

        lax.fori_loop(0, ngroups, group, ())
        last_even = ((ngroups - 1) % 2) == 0

        @pl.when(last_even)
        def _():
            drain_scatter(dst_a, rows_a)

        @pl.when(jnp.logical_not(last_even))
        def _():
            drain_scatter(dst_b, rows_b)

        @pl.when((c == 0) & (s == 0))
        def _():
            fire_r = pl.multiple_of(XTRA_ROW, 4)
            pltpu.sync_copy(src_hbm.at[pl.ds(fire_r, 4)], src_a)
            pltpu.sync_copy(dst_hbm.at[pl.ds(fire_r, 4)], dst_a)
            for j in range(4):
                pltpu.async_copy(g_hbm.at[src_a.at[j]], rows_a.at[j], sem)
            drain_scatter(dst_a, rows_a)

        plsc.subcore_barrier()
        off2 = pl.multiple_of(c * N + s * RPT, 8)
        pltpu.sync_copy(acc.at[pl.ds(off, RPT)], out_hbm.at[pl.ds(off2, RPT)])

        @pl.when(s == 0)
        def _():
            off3 = pl.multiple_of(c * N + TAIL_OFF, 8)
            pltpu.sync_copy(acc.at[pl.ds(TAIL_OFF, TAIL)],
                            out_hbm.at[pl.ds(off3, TAIL)])

    return agg_kernel


_sc_agg64 = _make_sc_agg(64)
_sc_agg32 = _make_sc_agg(32)


def _tc_matmul_body(x_ref, w_ref, o_ref):
    o_ref[...] = jnp.dot(x_ref[...], w_ref[...],
                         preferred_element_type=jnp.float32)


def _tc1a(x, W1):
    return pl.pallas_call(
        _tc_matmul_body,
        grid=(1,),
        in_specs=[
            pl.BlockSpec((N, 128), lambda i: (0, 0)),
            pl.BlockSpec((128, 64), lambda i: (0, 0)),
        ],
        out_specs=pl.BlockSpec((N, 64), lambda i: (0, 0)),
        out_shape=jax.ShapeDtypeStruct((N, 64), jnp.float32),
    )(x, W1)


def _tc1b_body(xw_ref, d0_ref, d1_ref, g_ref, dis_ref):
    deg = d0_ref[:, 0:1] + d1_ref[:, 0:1] + 1.0
    dis = lax.rsqrt(deg)
    dis_ref[...] = jnp.broadcast_to(dis, (N, 8))
    g_ref[...] = xw_ref[...] * dis


def _tc1b(xw, degp):
    return pl.pallas_call(
        _tc1b_body,
        grid=(1,),
        in_specs=[
            pl.BlockSpec((N, 64), lambda i: (0, 0)),
            pl.BlockSpec((N, 8), lambda i: (0, 0)),
            pl.BlockSpec((N, 8), lambda i: (1, 0)),
        ],
        out_specs=[
            pl.BlockSpec((N, 64), lambda i: (0, 0)),
            pl.BlockSpec((N, 8), lambda i: (0, 0)),
        ],
        out_shape=[
            jax.ShapeDtypeStruct((N, 64), jnp.float32),
            jax.ShapeDtypeStruct((N, 8), jnp.float32),
        ],
    )(xw, degp, degp)


def _tc2_body(a0_ref, a1_ref, dis_ref, b_ref, w_ref, g_ref):
    d = dis_ref[:, 0:1]
    h = jnp.maximum((a0_ref[...] + a1_ref[...]) * d + b_ref[...], 0.0)
    g_ref[...] = jnp.dot(h, w_ref[...], preferred_element_type=jnp.float32) * d


def _tc2(agg1, dis, b1r, W2):
    return pl.pallas_call(
        _tc2_body,
        grid=(1,),
        in_specs=[
            pl.BlockSpec((N, 64), lambda i: (0, 0)),
            pl.BlockSpec((N, 64), lambda i: (1, 0)),
            pl.BlockSpec((N, 8), lambda i: (0, 0)),
            pl.BlockSpec((1, 64), lambda i: (0, 0)),
            pl.BlockSpec((64, 32), lambda i: (0, 0)),
        ],
        out_specs=pl.BlockSpec((N, 32), lambda i: (0, 0)),
        out_shape=jax.ShapeDtypeStruct((N, 32), jnp.float32),
    )(agg1, agg1, dis, b1r, W2)


def _tc3_body(a0_ref, a1_ref, dis_ref, b2_ref, w_ref, b3_ref, o_ref):
    d = dis_ref[:, 0:1]
    h = jnp.maximum((a0_ref[...] + a1_ref[...]) * d + b2_ref[...], 0.0)
    o_ref[...] = jnp.dot(h, w_ref[...],
                         preferred_element_type=jnp.float32) + b3_ref[...]


def _tc3(agg2, dis, b2r, W3, b3r):
    return pl.pallas_call(
        _tc3_body,
        grid=(1,),
        in_specs=[
            pl.BlockSpec((N, 32), lambda i: (0, 0)),
            pl.BlockSpec((N, 32), lambda i: (1, 0)),
            pl.BlockSpec((N, 8), lambda i: (0, 0)),
            pl.BlockSpec((1, 32), lambda i: (0, 0)),
            pl.BlockSpec((32, 1), lambda i: (0, 0)),
            pl.BlockSpec((1, 1), lambda i: (0, 0)),
        ],
        out_specs=pl.BlockSpec((N, 1), lambda i: (0, 0)),
        out_shape=jax.ShapeDtypeStruct((N, 1), jnp.float32),
    )(agg2, agg2, dis, b2r, W3, b3r)


def kernel(x, edge_index, W1, b1, W2, b2, W3, b3):
    ei = edge_index.astype(jnp.int32)
    src = ei[0].reshape(IDX_ROWS, 128)
    dst = ei[1].reshape(IDX_ROWS, 128)
    ones = jnp.ones((128, 8), jnp.float32)
    z8 = jnp.zeros((N, 8), jnp.float32)
    z64 = jnp.zeros((N, 64), jnp.float32)
    z32 = jnp.zeros((N, 32), jnp.float32)

    degp = _sc_degree(dst, ones, z8)
    xw = _tc1a(x, W1)
    g1, dis = _tc1b(xw, degp)
    agg1 = _sc_agg64(g1, z64, src, dst)
    g2 = _tc2(agg1, dis, b1.reshape(1, 64), W2)
    agg2 = _sc_agg32(g2, z32, src, dst)
    out = _tc3(agg2, dis, b2.reshape(1, 32), W3, b3.reshape(1, 1))
    return out


# speedup vs baseline: 43.7972x; 1.0915x over previous
import functools

import jax
import jax.numpy as jnp
from jax import lax
from jax.experimental import pallas as pl
from jax.experimental.pallas import tpu as pltpu
from jax.experimental.pallas import tpu_sc as plsc

N = 10000
E = 320000
NC, NS = 2, 16
IDX_ROWS = E // 128
GRP = 4
R0, R1 = 80, 76
XTRA_ROW = NS * (R0 + R1)
RPT = 624
TAIL_OFF = NS * RPT
TAIL = N - TAIL_OFF
DEG_R = 78

_SC_PARAMS = pltpu.CompilerParams(use_tc_tiling_on_sc=False)


def _sc_degree(dstm, ones, zeros8):
    mesh = plsc.VectorSubcoreMesh(core_axis_name="c", subcore_axis_name="s")

    @functools.partial(
        pl.kernel,
        out_type=jax.ShapeDtypeStruct((2 * N, 8), jnp.float32),
        mesh=mesh,
        scratch_types=[
            pltpu.VMEM((GRP, 128), jnp.int32),
            pltpu.VMEM((GRP, 128), jnp.int32),
            pltpu.VMEM((128, 8), jnp.float32),
            pltpu.VMEM_SHARED((N + 16, 8), jnp.float32),
            pltpu.SemaphoreType.DMA,
            pltpu.SemaphoreType.DMA,
        ],
        compiler_params=_SC_PARAMS,
    )
    def deg_kernel(ei_hbm, ones_hbm, z_hbm, out_hbm,
                   dst_a, dst_b, ones_v, acc, sem, sem2):
        dst_hbm = ei_hbm.at[1]
        c = lax.axis_index("c")
        s = lax.axis_index("s")
        wid = s * NC + c
        off = pl.multiple_of(s * RPT, 8)
        pltpu.sync_copy(ones_hbm, ones_v)
        pltpu.sync_copy(z_hbm.at[pl.ds(off, RPT)], acc.at[pl.ds(off, RPT)])

        @pl.when(s == 0)
        def _():
            pltpu.sync_copy(z_hbm.at[pl.ds(TAIL_OFF, TAIL)],
                            acc.at[pl.ds(TAIL_OFF, TAIL)])

        plsc.subcore_barrier()

        row0 = wid * DEG_R

        def fire(gi, dst_v):
            r = pl.multiple_of(row0 + gi * GRP, 2)
            pltpu.async_copy(dst_hbm.at[pl.ds(r, GRP)], dst_v, sem)

        def wait_idx(dst_v):
            pltpu.make_async_copy(dst_hbm.at[pl.ds(0, GRP)], dst_v, sem).wait()

        def fire_scatters(dst_v, nrows):
            for j in range(nrows):
                pltpu.async_copy(ones_v, acc.at[dst_v.at[j]], sem2, add=True)

        def wait_scatters(nrows):
            for j in range(nrows):
                pltpu.make_async_copy(z_hbm.at[pl.ds(0, 128)],
                                      ones_v, sem2).wait()

        def group(gi, _):
            even = (gi % 2) == 0

            @pl.when(gi >= 2)
            def _():
                wait_scatters(GRP)

            @pl.when(even)
            def _():
                fire(gi, dst_a)

            @pl.when(jnp.logical_not(even))
            def _():
                fire(gi, dst_b)

            @pl.when(gi >= 1)
            def _():
                wait_idx(dst_a)

            @pl.when((gi >= 1) & even)
            def _():
                fire_scatters(dst_b, GRP)

            @pl.when((gi >= 1) & jnp.logical_not(even))
            def _():
                fire_scatters(dst_a, GRP)

            return ()

        lax.fori_loop(0, 19, group, ())
        wait_idx(dst_a)
        fire_scatters(dst_a, GRP)
        r2 = pl.multiple_of(row0 + 76, 2)
        pltpu.sync_copy(dst_hbm.at[pl.ds(r2, 2)], dst_b.at[pl.ds(0, 2)])
        fire_scatters(dst_b, 2)

        @pl.when((c == 0) & (s == 0))
        def _():
            pltpu.sync_copy(dst_hbm.at[pl.ds(XTRA_ROW, 4)], dst_a)
            fire_scatters(dst_a, 4)
            wait_scatters(4)

        wait_scatters(2 * GRP + 2)

        plsc.subcore_barrier()
        off2 = pl.multiple_of(c * N + s * RPT, 8)
        pltpu.sync_copy(acc.at[pl.ds(off, RPT)], out_hbm.at[pl.ds(off2, RPT)])

        @pl.when(s == 0)
        def _():
            off3 = pl.multiple_of(c * N + TAIL_OFF, 8)
            pltpu.sync_copy(acc.at[pl.ds(TAIL_OFF, TAIL)],
                            out_hbm.at[pl.ds(off3, TAIL)])

    return deg_kernel(dstm, ones, zeros8)


def _make_sc_agg(F):
    mesh = plsc.VectorSubcoreMesh(core_axis_name="c", subcore_axis_name="s")

    @functools.partial(
        pl.kernel,
        out_type=jax.ShapeDtypeStruct((2 * N, F), jnp.float32),
        mesh=mesh,
        scratch_types=[
            pltpu.VMEM((GRP, 128), jnp.int32),
            pltpu.VMEM((GRP, 128), jnp.int32),
            pltpu.VMEM((GRP, 128), jnp.int32),
            pltpu.VMEM((GRP, 128), jnp.int32),
            pltpu.VMEM((GRP, 128, F), jnp.float32),
            pltpu.VMEM((GRP, 128, F), jnp.float32),
            pltpu.VMEM_SHARED((N + 16, F), jnp.float32),
            pltpu.SemaphoreType.DMA,
            pltpu.SemaphoreType.DMA,
        ],
        compiler_params=_SC_PARAMS,
    )
    def agg_kernel(g_hbm, z_hbm, ei_hbm, out_hbm,
                   src_a, dst_a, src_b, dst_b, rows_a, rows_b, acc,
                   sem, sem2):
        src_hbm = ei_hbm.at[0]
        dst_hbm = ei_hbm.at[1]
        c = lax.axis_index("c")
        s = lax.axis_index("s")
        off = pl.multiple_of(s * RPT, 8)

        @pl.when(c == 0)
        def _():
            pltpu.sync_copy(g_hbm.at[pl.ds(off, RPT)], acc.at[pl.ds(off, RPT)])

            @pl.when(s == 0)
            def _():
                pltpu.sync_copy(g_hbm.at[pl.ds(TAIL_OFF, TAIL)],
                                acc.at[pl.ds(TAIL_OFF, TAIL)])

        @pl.when(c != 0)
        def _():
            pltpu.sync_copy(z_hbm.at[pl.ds(off, RPT)], acc.at[pl.ds(off, RPT)])

            @pl.when(s == 0)
            def _():
                pltpu.sync_copy(z_hbm.at[pl.ds(TAIL_OFF, TAIL)],
                                acc.at[pl.ds(TAIL_OFF, TAIL)])

        plsc.subcore_barrier()

        row0 = jnp.where(c == 0, s * R0, NS * R0 + s * R1)
        ngroups = jnp.where(c == 0, R0 // GRP, R1 // GRP)

        def fire(gi, src_v, dst_v, rows_v):
            r = pl.multiple_of(row0 + gi * GRP, 4)
            pltpu.sync_copy(src_hbm.at[pl.ds(r, GRP)], src_v)
            pltpu.sync_copy(dst_hbm.at[pl.ds(r, GRP)], dst_v)
            for j in range(GRP):
                pltpu.async_copy(g_hbm.at[src_v.at[j]], rows_v.at[j], sem)

        def wait_gathers(rows_v):
            for j in range(GRP):
                pltpu.make_async_copy(g_hbm.at[pl.ds(0, 128)],
                                      rows_v.at[j], sem).wait()

        def fire_scatters(dst_v, rows_v):
            for j in range(GRP):
                pltpu.async_copy(rows_v.at[j], acc.at[dst_v.at[j]], sem2,
                                 add=True)

        def wait_scatters(rows_v):
            for j in range(GRP):
                pltpu.make_async_copy(g_hbm.at[pl.ds(0, 128)],
                                      rows_v.at[j], sem2).wait()

        def group(gi, _):
            even = (gi % 2) == 0

            @pl.when(gi >= 2)
            def _():
                wait_scatters(rows_a)

            @pl.when(even)
            def _():
                fire(gi, src_a, dst_a, rows_a)

            @pl.when(jnp.logical_not(even))
            def _():
                fire(gi, src_b, dst_b, rows_b)

            @pl.when(gi >= 1)
            def _():
                wait_gathers(rows_a)

            @pl.when((gi >= 1) & even)
            def _():
                fire_scatters(dst_b, rows_b)

            @pl.when((gi >= 1) & jnp.logical_not(even))
            def _():
                fire_scatters(dst_a, rows_a)

            return ()

        lax.fori_loop(0, ngroups, group, ())
        last_even = ((ngroups - 1) % 2) == 0
        wait_gathers(rows_a)

        @pl.when(last_even)
        def _():
            fire_scatters(dst_a, rows_a)

        @pl.when(jnp.logical_not(last_even))
        def _():
            fire_scatters(dst_b, rows_b)

        wait_scatters(rows_a)
        wait_scatters(rows_b)

        @pl.when((c == 0) & (s == 0))
        def _():
            fire_r = pl.multiple_of(XTRA_ROW, 4)
            pltpu.sync_copy(src_hbm.at[pl.ds(fire_r, 4)], src_a)
            pltpu.sync_copy(dst_hbm.at[pl.ds(fire_r, 4)], dst_a)
            for j in range(4):
                pltpu.async_copy(g_hbm.at[src_a.at[j]], rows_a.at[j], sem)
            wait_gathers(rows_a)
            for j in range(4):
                pltpu.sync_copy(rows_a.at[j], acc.at[dst_a.at[j]], add=True)

        plsc.subcore_barrier()
        off2 = pl.multiple_of(c * N + s * RPT, 8)
        pltpu.sync_copy(acc.at[pl.ds(off, RPT)], out_hbm.at[pl.ds(off2, RPT)])

        @pl.when(s == 0)
        def _():
            off3 = pl.multiple_of(c * N + TAIL_OFF, 8)
            pltpu.sync_copy(acc.at[pl.ds(TAIL_OFF, TAIL)],
                            out_hbm.at[pl.ds(off3, TAIL)])

    return agg_kernel


_sc_agg64 = _make_sc_agg(64)
_sc_agg32 = _make_sc_agg(32)


def _tc_matmul_body(x_ref, w_ref, o_ref):
    o_ref[...] = jnp.dot(x_ref[...], w_ref[...],
                         preferred_element_type=jnp.float32)


def _tc1a(x, W1):
    return pl.pallas_call(
        _tc_matmul_body,
        grid=(1,),
        in_specs=[
            pl.BlockSpec((N, 128), lambda i: (0, 0)),
            pl.BlockSpec((128, 64), lambda i: (0, 0)),
        ],
        out_specs=pl.BlockSpec((N, 64), lambda i: (0, 0)),
        out_shape=jax.ShapeDtypeStruct((N, 64), jnp.float32),
    )(x, W1)


def _tc1b_body(xw_ref, d0_ref, d1_ref, g_ref, dis_ref):
    deg = d0_ref[:, 0:1] + d1_ref[:, 0:1] + 1.0
    dis = lax.rsqrt(deg)
    dis_ref[...] = jnp.broadcast_to(dis, (N, 8))
    g_ref[...] = xw_ref[...] * dis


def _tc1b(xw, degp):
    return pl.pallas_call(
        _tc1b_body,
        grid=(1,),
        in_specs=[
            pl.BlockSpec((N, 64), lambda i: (0, 0)),
            pl.BlockSpec((N, 8), lambda i: (0, 0)),
            pl.BlockSpec((N, 8), lambda i: (1, 0)),
        ],
        out_specs=[
            pl.BlockSpec((N, 64), lambda i: (0, 0)),
            pl.BlockSpec((N, 8), lambda i: (0, 0)),
        ],
        out_shape=[
            jax.ShapeDtypeStruct((N, 64), jnp.float32),
            jax.ShapeDtypeStruct((N, 8), jnp.float32),
        ],
    )(xw, degp, degp)


def _tc2_body(a0_ref, a1_ref, dis_ref, b_ref, w_ref, g_ref):
    d = dis_ref[:, 0:1]
    h = jnp.maximum((a0_ref[...] + a1_ref[...]) * d + b_ref[...], 0.0)
    g_ref[...] = jnp.dot(h, w_ref[...], preferred_element_type=jnp.float32) * d


def _tc2(agg1, dis, b1r, W2):
    return pl.pallas_call(
        _tc2_body,
        grid=(1,),
        in_specs=[
            pl.BlockSpec((N, 64), lambda i: (0, 0)),
            pl.BlockSpec((N, 64), lambda i: (1, 0)),
            pl.BlockSpec((N, 8), lambda i: (0, 0)),
            pl.BlockSpec((1, 64), lambda i: (0, 0)),
            pl.BlockSpec((64, 32), lambda i: (0, 0)),
        ],
        out_specs=pl.BlockSpec((N, 32), lambda i: (0, 0)),
        out_shape=jax.ShapeDtypeStruct((N, 32), jnp.float32),
    )(agg1, agg1, dis, b1r, W2)


def _tc3_body(a0_ref, a1_ref, dis_ref, b2_ref, w_ref, b3_ref, o_ref):
    d = dis_ref[:, 0:1]
    h = jnp.maximum((a0_ref[...] + a1_ref[...]) * d + b2_ref[...], 0.0)
    o_ref[...] = jnp.dot(h, w_ref[...],
                         preferred_element_type=jnp.float32) + b3_ref[...]


def _tc3(agg2, dis, b2r, W3, b3r):
    return pl.pallas_call(
        _tc3_body,
        grid=(1,),
        in_specs=[
            pl.BlockSpec((N, 32), lambda i: (0, 0)),
            pl.BlockSpec((N, 32), lambda i: (1, 0)),
            pl.BlockSpec((N, 8), lambda i: (0, 0)),
            pl.BlockSpec((1, 32), lambda i: (0, 0)),
            pl.BlockSpec((32, 1), lambda i: (0, 0)),
            pl.BlockSpec((1, 1), lambda i: (0, 0)),
        ],
        out_specs=pl.BlockSpec((N, 1), lambda i: (0, 0)),
        out_shape=jax.ShapeDtypeStruct((N, 1), jnp.float32),
    )(agg2, agg2, dis, b2r, W3, b3r)


def kernel(x, edge_index, W1, b1, W2, b2, W3, b3):
    ei3 = edge_index.astype(jnp.int32).reshape(2, IDX_ROWS, 128)
    ones = jnp.ones((128, 8), jnp.float32)
    z8 = jnp.zeros((N, 8), jnp.float32)
    z64 = jnp.zeros((N, 64), jnp.float32)
    z32 = jnp.zeros((N, 32), jnp.float32)

    degp = _sc_degree(ei3, ones, z8)
    xw = _tc1a(x, W1)
    g1, dis = _tc1b(xw, degp)
    agg1 = _sc_agg64(g1, z64, ei3)
    g2 = _tc2(agg1, dis, b1.reshape(1, 64), W2)
    agg2 = _sc_agg32(g2, z32, ei3)
    out = _tc3(agg2, dis, b2.reshape(1, 32), W3, b3.reshape(1, 1))
    return out


# speedup vs baseline: 44.9805x; 1.0270x over previous
import functools

import jax
import jax.numpy as jnp
from jax import lax
from jax.experimental import pallas as pl
from jax.experimental.pallas import tpu as pltpu
from jax.experimental.pallas import tpu_sc as plsc

N = 10000
E = 320000
NC, NS = 2, 16
IDX_ROWS = E // 128
GRP = 4
XTRA_ROW = 2496
RPT = 624
TAIL_OFF = NS * RPT
TAIL = N - TAIL_OFF
DEG_R = 78

_SC_PARAMS = pltpu.CompilerParams(use_tc_tiling_on_sc=False)


def _sc_degree(dstm, ones, zeros8):
    mesh = plsc.VectorSubcoreMesh(core_axis_name="c", subcore_axis_name="s")

    @functools.partial(
        pl.kernel,
        out_type=jax.ShapeDtypeStruct((2 * N, 8), jnp.float32),
        mesh=mesh,
        scratch_types=[
            pltpu.VMEM((GRP, 128), jnp.int32),
            pltpu.VMEM((GRP, 128), jnp.int32),
            pltpu.VMEM((128, 8), jnp.float32),
            pltpu.VMEM_SHARED((N + 16, 8), jnp.float32),
            pltpu.SemaphoreType.DMA,
            pltpu.SemaphoreType.DMA,
        ],
        compiler_params=_SC_PARAMS,
    )
    def deg_kernel(ei_hbm, ones_hbm, z_hbm, out_hbm,
                   dst_a, dst_b, ones_v, acc, sem, sem2):
        dst_hbm = ei_hbm.at[1]
        c = lax.axis_index("c")
        s = lax.axis_index("s")
        wid = s * NC + c
        off = pl.multiple_of(s * RPT, 8)
        pltpu.sync_copy(ones_hbm, ones_v)
        pltpu.sync_copy(z_hbm.at[pl.ds(off, RPT)], acc.at[pl.ds(off, RPT)])

        @pl.when(s == 0)
        def _():
            pltpu.sync_copy(z_hbm.at[pl.ds(TAIL_OFF, TAIL)],
                            acc.at[pl.ds(TAIL_OFF, TAIL)])

        plsc.subcore_barrier()

        row0 = wid * DEG_R

        def fire(gi, dst_v):
            r = pl.multiple_of(row0 + gi * GRP, 2)
            pltpu.async_copy(dst_hbm.at[pl.ds(r, GRP)], dst_v, sem)

        def wait_idx(dst_v):
            pltpu.make_async_copy(dst_hbm.at[pl.ds(0, GRP)], dst_v, sem).wait()

        def fire_scatters(dst_v, nrows):
            for j in range(nrows):
                pltpu.async_copy(ones_v, acc.at[dst_v.at[j]], sem2, add=True)

        def wait_scatters(nrows):
            for j in range(nrows):
                pltpu.make_async_copy(z_hbm.at[pl.ds(0, 128)],
                                      ones_v, sem2).wait()

        def group(gi, _):
            even = (gi % 2) == 0

            @pl.when(gi >= 2)
            def _():
                wait_scatters(GRP)

            @pl.when(even)
            def _():
                fire(gi, dst_a)

            @pl.when(jnp.logical_not(even))
            def _():
                fire(gi, dst_b)

            @pl.when(gi >= 1)
            def _():
                wait_idx(dst_a)

            @pl.when((gi >= 1) & even)
            def _():
                fire_scatters(dst_b, GRP)

            @pl.when((gi >= 1) & jnp.logical_not(even))
            def _():
                fire_scatters(dst_a, GRP)

            return ()

        lax.fori_loop(0, 19, group, ())
        wait_idx(dst_a)
        fire_scatters(dst_a, GRP)
        r2 = pl.multiple_of(row0 + 76, 2)
        pltpu.sync_copy(dst_hbm.at[pl.ds(r2, 2)], dst_b.at[pl.ds(0, 2)])
        fire_scatters(dst_b, 2)

        @pl.when((c == 0) & (s == 0))
        def _():
            pltpu.sync_copy(dst_hbm.at[pl.ds(XTRA_ROW, 4)], dst_a)
            fire_scatters(dst_a, 4)
            wait_scatters(4)

        wait_scatters(2 * GRP + 2)

        plsc.subcore_barrier()
        off2 = pl.multiple_of(c * N + s * RPT, 8)
        pltpu.sync_copy(acc.at[pl.ds(off, RPT)], out_hbm.at[pl.ds(off2, RPT)])

        @pl.when(s == 0)
        def _():
            off3 = pl.multiple_of(c * N + TAIL_OFF, 8)
            pltpu.sync_copy(acc.at[pl.ds(TAIL_OFF, TAIL)],
                            out_hbm.at[pl.ds(off3, TAIL)])

    return deg_kernel(dstm, ones, zeros8)


def _make_sc_agg(F, AGRP, R0, R1):
    mesh = plsc.VectorSubcoreMesh(core_axis_name="c", subcore_axis_name="s")

    @functools.partial(
        pl.kernel,
        out_type=jax.ShapeDtypeStruct((2 * N, F), jnp.float32),
        mesh=mesh,
        scratch_types=[
            pltpu.VMEM((AGRP, 128), jnp.int32),
            pltpu.VMEM((AGRP, 128), jnp.int32),
            pltpu.VMEM((AGRP, 128), jnp.int32),
            pltpu.VMEM((AGRP, 128), jnp.int32),
            pltpu.VMEM((AGRP, 128, F), jnp.float32),
            pltpu.VMEM((AGRP, 128, F), jnp.float32),
            pltpu.VMEM_SHARED((N + 16, F), jnp.float32),
            pltpu.SemaphoreType.DMA,
            pltpu.SemaphoreType.DMA,
        ],
        compiler_params=_SC_PARAMS,
    )
    def agg_kernel(g_hbm, z_hbm, ei_hbm, out_hbm,
                   src_a, dst_a, src_b, dst_b, rows_a, rows_b, acc,
                   sem, sem2):
        src_hbm = ei_hbm.at[0]
        dst_hbm = ei_hbm.at[1]
        c = lax.axis_index("c")
        s = lax.axis_index("s")
        off = pl.multiple_of(s * RPT, 8)

        @pl.when(c == 0)
        def _():
            pltpu.sync_copy(g_hbm.at[pl.ds(off, RPT)], acc.at[pl.ds(off, RPT)])

            @pl.when(s == 0)
            def _():
                pltpu.sync_copy(g_hbm.at[pl.ds(TAIL_OFF, TAIL)],
                                acc.at[pl.ds(TAIL_OFF, TAIL)])

        @pl.when(c != 0)
        def _():
            pltpu.sync_copy(z_hbm.at[pl.ds(off, RPT)], acc.at[pl.ds(off, RPT)])

            @pl.when(s == 0)
            def _():
                pltpu.sync_copy(z_hbm.at[pl.ds(TAIL_OFF, TAIL)],
                                acc.at[pl.ds(TAIL_OFF, TAIL)])

        plsc.subcore_barrier()

        row0 = jnp.where(c == 0, s * R0, NS * R0 + s * R1)
        ngroups = jnp.where(c == 0, R0 // AGRP, R1 // AGRP)

        def fire(gi, src_v, dst_v, rows_v):
            r = pl.multiple_of(row0 + gi * AGRP, 2)
            pltpu.sync_copy(src_hbm.at[pl.ds(r, AGRP)], src_v)
            pltpu.sync_copy(dst_hbm.at[pl.ds(r, AGRP)], dst_v)
            for j in range(AGRP):
                pltpu.async_copy(g_hbm.at[src_v.at[j]], rows_v.at[j], sem)

        def wait_gathers(rows_v):
            for j in range(AGRP):
                pltpu.make_async_copy(g_hbm.at[pl.ds(0, 128)],
                                      rows_v.at[j], sem).wait()

        def fire_scatters(dst_v, rows_v):
            for j in range(AGRP):
                pltpu.async_copy(rows_v.at[j], acc.at[dst_v.at[j]], sem2,
                                 add=True)

        def wait_scatters(rows_v):
            for j in range(AGRP):
                pltpu.make_async_copy(g_hbm.at[pl.ds(0, 128)],
                                      rows_v.at[j], sem2).wait()

        def group(gi, _):
            even = (gi % 2) == 0

            @pl.when(gi >= 2)
            def _():
                wait_scatters(rows_a)

            @pl.when(even)
            def _():
                fire(gi, src_a, dst_a, rows_a)

            @pl.when(jnp.logical_not(even))
            def _():
                fire(gi, src_b, dst_b, rows_b)

            @pl.when(gi >= 1)
            def _():
                wait_gathers(rows_a)

            @pl.when((gi >= 1) & even)
            def _():
                fire_scatters(dst_b, rows_b)

            @pl.when((gi >= 1) & jnp.logical_not(even))
            def _():
                fire_scatters(dst_a, rows_a)

            return ()

        lax.fori_loop(0, ngroups, group, ())
        last_even = ((ngroups - 1) % 2) == 0
        wait_gathers(rows_a)

        @pl.when(last_even)
        def _():
            fire_scatters(dst_a, rows_a)

        @pl.when(jnp.logical_not(last_even))
        def _():
            fire_scatters(dst_b, rows_b)

        wait_scatters(rows_a)
        wait_scatters(rows_b)

        @pl.when((c == 1) & (s == 0))
        def _():
            fire_r = pl.multiple_of(XTRA_ROW, 4)
            pltpu.sync_copy(src_hbm.at[pl.ds(fire_r, 4)], src_a.at[pl.ds(0, 4)])
            pltpu.sync_copy(dst_hbm.at[pl.ds(fire_r, 4)], dst_a.at[pl.ds(0, 4)])
            for j in range(4):
                pltpu.async_copy(g_hbm.at[src_a.at[j]], rows_a.at[j], sem)
            for j in range(4):
                pltpu.make_async_copy(g_hbm.at[pl.ds(0, 128)],
                                      rows_a.at[j], sem).wait()
            for j in range(4):
                pltpu.sync_copy(rows_a.at[j], acc.at[dst_a.at[j]], add=True)

        plsc.subcore_barrier()
        off2 = pl.multiple_of(c * N + s * RPT, 8)
        pltpu.sync_copy(acc.at[pl.ds(off, RPT)], out_hbm.at[pl.ds(off2, RPT)])

        @pl.when(s == 0)
        def _():
            off3 = pl.multiple_of(c * N + TAIL_OFF, 8)
            pltpu.sync_copy(acc.at[pl.ds(TAIL_OFF, TAIL)],
                            out_hbm.at[pl.ds(off3, TAIL)])

    return agg_kernel


_sc_agg64 = _make_sc_agg(64, 4, 76, 80)
_sc_agg32 = _make_sc_agg(32, 6, 78, 78)


def _tc_matmul_body(x_ref, w_ref, o_ref):
    o_ref[...] = jnp.dot(x_ref[...], w_ref[...],
                         preferred_element_type=jnp.float32)


def _tc1a(x, W1):
    return pl.pallas_call(
        _tc_matmul_body,
        grid=(1,),
        in_specs=[
            pl.BlockSpec((N, 128), lambda i: (0, 0)),
            pl.BlockSpec((128, 64), lambda i: (0, 0)),
        ],
        out_specs=pl.BlockSpec((N, 64), lambda i: (0, 0)),
        out_shape=jax.ShapeDtypeStruct((N, 64), jnp.float32),
    )(x, W1)


def _tc1b_body(xw_ref, d0_ref, d1_ref, g_ref, dis_ref):
    deg = d0_ref[:, 0:1] + d1_ref[:, 0:1] + 1.0
    dis = lax.rsqrt(deg)
    dis_ref[...] = jnp.broadcast_to(dis, (N, 8))
    g_ref[...] = xw_ref[...] * dis


def _tc1b(xw, degp):
    return pl.pallas_call(
        _tc1b_body,
        grid=(1,),
        in_specs=[
            pl.BlockSpec((N, 64), lambda i: (0, 0)),
            pl.BlockSpec((N, 8), lambda i: (0, 0)),
            pl.BlockSpec((N, 8), lambda i: (1, 0)),
        ],
        out_specs=[
            pl.BlockSpec((N, 64), lambda i: (0, 0)),
            pl.BlockSpec((N, 8), lambda i: (0, 0)),
        ],
        out_shape=[
            jax.ShapeDtypeStruct((N, 64), jnp.float32),
            jax.ShapeDtypeStruct((N, 8), jnp.float32),
        ],
    )(xw, degp, degp)


def _tc2_body(a0_ref, a1_ref, dis_ref, b_ref, w_ref, g_ref):
    d = dis_ref[:, 0:1]
    h = jnp.maximum((a0_ref[...] + a1_ref[...]) * d + b_ref[...], 0.0)
    g_ref[...] = jnp.dot(h, w_ref[...], preferred_element_type=jnp.float32) * d


def _tc2(agg1, dis, b1r, W2):
    return pl.pallas_call(
        _tc2_body,
        grid=(1,),
        in_specs=[
            pl.BlockSpec((N, 64), lambda i: (0, 0)),
            pl.BlockSpec((N, 64), lambda i: (1, 0)),
            pl.BlockSpec((N, 8), lambda i: (0, 0)),
            pl.BlockSpec((1, 64), lambda i: (0, 0)),
            pl.BlockSpec((64, 32), lambda i: (0, 0)),
        ],
        out_specs=pl.BlockSpec((N, 32), lambda i: (0, 0)),
        out_shape=jax.ShapeDtypeStruct((N, 32), jnp.float32),
    )(agg1, agg1, dis, b1r, W2)


def _tc3_body(a0_ref, a1_ref, dis_ref, b2_ref, w_ref, b3_ref, o_ref):
    d = dis_ref[:, 0:1]
    h = jnp.maximum((a0_ref[...] + a1_ref[...]) * d + b2_ref[...], 0.0)
    o_ref[...] = jnp.dot(h, w_ref[...],
                         preferred_element_type=jnp.float32) + b3_ref[...]


def _tc3(agg2, dis, b2r, W3, b3r):
    return pl.pallas_call(
        _tc3_body,
        grid=(1,),
        in_specs=[
            pl.BlockSpec((N, 32), lambda i: (0, 0)),
            pl.BlockSpec((N, 32), lambda i: (1, 0)),
            pl.BlockSpec((N, 8), lambda i: (0, 0)),
            pl.BlockSpec((1, 32), lambda i: (0, 0)),
            pl.BlockSpec((32, 1), lambda i: (0, 0)),
            pl.BlockSpec((1, 1), lambda i: (0, 0)),
        ],
        out_specs=pl.BlockSpec((N, 1), lambda i: (0, 0)),
        out_shape=jax.ShapeDtypeStruct((N, 1), jnp.float32),
    )(agg2, agg2, dis, b2r, W3, b3r)


def kernel(x, edge_index, W1, b1, W2, b2, W3, b3):
    ei3 = edge_index.astype(jnp.int32).reshape(2, IDX_ROWS, 128)
    ones = jnp.ones((128, 8), jnp.float32)
    z8 = jnp.zeros((N, 8), jnp.float32)
    z64 = jnp.zeros((N, 64), jnp.float32)
    z32 = jnp.zeros((N, 32), jnp.float32)

    degp = _sc_degree(ei3, ones, z8)
    xw = _tc1a(x, W1)
    g1, dis = _tc1b(xw, degp)
    agg1 = _sc_agg64(g1, z64, ei3)
    g2 = _tc2(agg1, dis, b1.reshape(1, 64), W2)
    agg2 = _sc_agg32(g2, z32, ei3)
    out = _tc3(agg2, dis, b2.reshape(1, 32), W3, b3.reshape(1, 1))
    return out
